# baseline (speedup 1.0000x reference)
import functools
import jax
import jax.numpy as jnp
from jax import lax
from jax.experimental import pallas as pl
from jax.experimental.pallas import tpu as pltpu
from jax.experimental.pallas import tpu_sc as plsc

EMBED_DIM = 64
PAIR_DIM = 2 * EMBED_DIM
NUM_CORES = 2
NUM_SUBCORES = 16
NUM_WORKERS = NUM_CORES * NUM_SUBCORES
CHUNK = 512
SEL_ROWS = 16


PACK_COLS = 2048
PACK_SHIFT = PACK_COLS.bit_length() - 1


def _tc_pack(weight_t, vocab):
    num_blocks = pl.cdiv(vocab, 2 * PACK_COLS)
    num_pairs = num_blocks * PACK_COLS

    def pack_kernel(wt_ref, out_ref):
        out_ref[:, :EMBED_DIM] = wt_ref[:, :PACK_COLS].T
        out_ref[:, EMBED_DIM:] = wt_ref[:, PACK_COLS:].T

    grid = (num_blocks,)
    return pl.pallas_call(
        pack_kernel,
        grid=grid,
        in_specs=[pl.BlockSpec((EMBED_DIM, 2 * PACK_COLS),
                               lambda i: (0, i))],
        out_specs=pl.BlockSpec((PACK_COLS, PAIR_DIM), lambda i: (i, 0)),
        out_shape=jax.ShapeDtypeStruct((num_pairs, PAIR_DIM), jnp.float32),
    )(weight_t)


def _sc_gather(table2, idx2, num_indices, chunk=CHUNK):
    mesh = plsc.VectorSubcoreMesh(core_axis_name="c", subcore_axis_name="s")
    per_worker = num_indices // NUM_WORKERS
    num_chunks = per_worker // chunk

    @functools.partial(
        pl.kernel,
        mesh=mesh,
        out_type=jax.ShapeDtypeStruct((num_indices, PAIR_DIM), jnp.float32),
        scratch_types=[
            pltpu.VMEM((chunk,), jnp.int32),
            pltpu.VMEM((chunk, PAIR_DIM), jnp.float32),
            pltpu.SemaphoreType.DMA,
        ],
    )
    def sc_kernel(table_hbm, idx_hbm, out_hbm, idx_v, rows_v, sem):
        wid = lax.axis_index("s") * NUM_CORES + lax.axis_index("c")
        base = wid * per_worker

        @pl.loop(0, num_chunks)
        def _(c):
            start = base + c * chunk
            pltpu.sync_copy(idx_hbm.at[pl.ds(start, chunk)], idx_v)
            pltpu.async_copy(table_hbm.at[idx_v], rows_v, sem).wait()
            pltpu.sync_copy(rows_v, out_hbm.at[pl.ds(start, chunk)])

    return sc_kernel(table2, idx2)


def _tc_select(pairs, idx_lanes, num_rows, seq):
    rows_per_step = SEL_ROWS * seq

    lane_rows = rows_per_step // 128

    def sel_kernel(pairs_ref, idx_ref, out_ref):
        pm = (lax.shift_right_logical(idx_ref[...], PACK_SHIFT) & 1).astype(
            jnp.float32)
        pm3 = pm.reshape(lane_rows, 128, 1)
        pairs3 = pairs_ref[...].reshape(lane_rows, 128, PAIR_DIM)
        sel = (pairs3[:, :, :EMBED_DIM] * (1.0 - pm3)
               + pairs3[:, :, EMBED_DIM:] * pm3)
        out_ref[...] = sel.reshape(rows_per_step, EMBED_DIM)

    grid = (num_rows // rows_per_step,)
    return pl.pallas_call(
        sel_kernel,
        grid=grid,
        in_specs=[
            pl.BlockSpec((rows_per_step, PAIR_DIM), lambda i: (i, 0)),
            pl.BlockSpec((1, rows_per_step // 128, 128),
                         lambda i: (i, 0, 0)),
        ],
        out_specs=pl.BlockSpec((rows_per_step, EMBED_DIM),
                               lambda i: (i, 0)),
        out_shape=jax.ShapeDtypeStruct((num_rows, EMBED_DIM),
                                       jnp.float32),
    )(pairs, idx_lanes)


NUM_SEGMENTS = 4


def kernel(batchinput, weight):
    batch, seq = batchinput.shape
    num_indices = batch * seq
    idx_flat = batchinput.reshape(num_indices)
    idx2 = (lax.shift_left(lax.shift_right_logical(idx_flat, PACK_SHIFT + 1),
                           PACK_SHIFT)
            | (idx_flat & (PACK_COLS - 1)))
    rows_per_step = SEL_ROWS * seq

    table2 = _tc_pack(weight.T, weight.shape[0])

    seg = num_indices // NUM_SEGMENTS
    seg_chunk = seg // NUM_WORKERS
    while seg_chunk * PAIR_DIM * 4 > 220 * 1024:
        seg_chunk //= 2
    outs = []
    for k in range(NUM_SEGMENTS):
        idx2_k = lax.slice(idx2, (k * seg,), ((k + 1) * seg,))
        idx_lanes_k = idx_flat[k * seg:(k + 1) * seg].reshape(
            seg // rows_per_step, rows_per_step // 128, 128)
        pairs_k = _sc_gather(table2, idx2_k, seg, chunk=seg_chunk)
        outs.append(_tc_select(pairs_k, idx_lanes_k, seg, seq))
    out = jnp.concatenate(outs, axis=0)
    return out.reshape(batch, seq, EMBED_DIM)


# speedup vs baseline: 1.4831x; 1.4831x over previous
import functools
import jax
import jax.numpy as jnp
from jax import lax
from jax.experimental import pallas as pl
from jax.experimental.pallas import tpu as pltpu
from jax.experimental.pallas import tpu_sc as plsc

EMBED_DIM = 64
ROW_LANES = 128
NUM_CORES = 2
NUM_SUBCORES = 16
NUM_WORKERS = NUM_CORES * NUM_SUBCORES
CHUNK = 512
PACK_COLS = 2048


def _tc_pack(weight_t, vocab):
    num_blocks = pl.cdiv(vocab, PACK_COLS)

    def pack_kernel(wt_ref, out_ref):
        out_ref[:, :EMBED_DIM] = wt_ref[...].T

    return pl.pallas_call(
        pack_kernel,
        grid=(num_blocks,),
        in_specs=[pl.BlockSpec((EMBED_DIM, PACK_COLS), lambda i: (0, i))],
        out_specs=pl.BlockSpec((PACK_COLS, ROW_LANES), lambda i: (i, 0)),
        out_shape=jax.ShapeDtypeStruct((num_blocks * PACK_COLS, ROW_LANES),
                                       jnp.float32),
    )(weight_t)


def _sc_gather(table, idx, num_indices, chunk=CHUNK):
    mesh = plsc.VectorSubcoreMesh(core_axis_name="c", subcore_axis_name="s")
    per_worker = num_indices // NUM_WORKERS
    num_chunks = per_worker // chunk

    @functools.partial(
        pl.kernel,
        mesh=mesh,
        out_type=jax.ShapeDtypeStruct((num_indices, ROW_LANES), jnp.float32),
        scratch_types=[
            pltpu.VMEM((chunk,), jnp.int32),
            pltpu.VMEM((chunk, ROW_LANES), jnp.float32),
            pltpu.SemaphoreType.DMA,
        ],
    )
    def sc_kernel(table_hbm, idx_hbm, out_hbm, idx_v, rows_v, sem):
        wid = lax.axis_index("s") * NUM_CORES + lax.axis_index("c")
        base = wid * per_worker

        @pl.loop(0, num_chunks)
        def _(c):
            start = base + c * chunk
            pltpu.sync_copy(idx_hbm.at[pl.ds(start, chunk)], idx_v)
            pltpu.async_copy(table_hbm.at[idx_v], rows_v, sem).wait()
            pltpu.sync_copy(rows_v, out_hbm.at[pl.ds(start, chunk)])

    return sc_kernel(table, idx)


def kernel(batchinput, weight):
    batch, seq = batchinput.shape
    num_indices = batch * seq
    idx_flat = batchinput.reshape(num_indices)

    table = _tc_pack(weight.T, weight.shape[0])
    rows = _sc_gather(table, idx_flat, num_indices)
    return rows[:, :EMBED_DIM].reshape(batch, seq, EMBED_DIM)


# speedup vs baseline: 1.8529x; 1.2494x over previous
import functools
import jax
import jax.numpy as jnp
from jax import lax
from jax.experimental import pallas as pl
from jax.experimental.pallas import tpu as pltpu
from jax.experimental.pallas import tpu_sc as plsc

EMBED_DIM = 64
ROW_LANES = 128
NUM_CORES = 2
NUM_SUBCORES = 16
NUM_WORKERS = NUM_CORES * NUM_SUBCORES
CHUNK = 512
PACK_COLS = 8192


def _tc_pack(weight_t, vocab):
    num_blocks = pl.cdiv(vocab, PACK_COLS)

    def pack_kernel(wt_ref, out_ref):
        out_ref[:, :EMBED_DIM] = wt_ref[...].T

    return pl.pallas_call(
        pack_kernel,
        grid=(num_blocks,),
        in_specs=[pl.BlockSpec((EMBED_DIM, PACK_COLS), lambda i: (0, i))],
        out_specs=pl.BlockSpec((PACK_COLS, ROW_LANES), lambda i: (i, 0)),
        out_shape=jax.ShapeDtypeStruct((num_blocks * PACK_COLS, ROW_LANES),
                                       jnp.float32),
    )(weight_t)


def _sc_gather(table, idx, num_indices, chunk=CHUNK):
    mesh = plsc.VectorSubcoreMesh(core_axis_name="c", subcore_axis_name="s")
    per_worker = num_indices // NUM_WORKERS
    num_chunks = per_worker // chunk

    @functools.partial(
        pl.kernel,
        mesh=mesh,
        out_type=jax.ShapeDtypeStruct((num_indices, ROW_LANES), jnp.float32),
        scratch_types=[
            pltpu.VMEM((chunk,), jnp.int32),
            pltpu.VMEM((chunk, ROW_LANES), jnp.float32),
            pltpu.SemaphoreType.DMA,
        ],
    )
    def sc_kernel(table_hbm, idx_hbm, out_hbm, idx_v, rows_v, sem):
        wid = lax.axis_index("s") * NUM_CORES + lax.axis_index("c")
        base = wid * per_worker

        @pl.loop(0, num_chunks)
        def _(c):
            start = base + c * chunk
            pltpu.sync_copy(idx_hbm.at[pl.ds(start, chunk)], idx_v)
            pltpu.async_copy(table_hbm.at[idx_v], rows_v, sem).wait()
            pltpu.sync_copy(rows_v, out_hbm.at[pl.ds(start, chunk)])

    return sc_kernel(table, idx)


def kernel(batchinput, weight):
    batch, seq = batchinput.shape
    num_indices = batch * seq
    idx_flat = batchinput.reshape(num_indices)

    table = _tc_pack(weight.T, weight.shape[0])
    rows = _sc_gather(table, idx_flat, num_indices)
    return rows[:, :EMBED_DIM].reshape(batch, seq, EMBED_DIM)


# speedup vs baseline: 1.8632x; 1.0055x over previous
import functools
import jax
import jax.numpy as jnp
from jax import lax
from jax.experimental import pallas as pl
from jax.experimental.pallas import tpu as pltpu
from jax.experimental.pallas import tpu_sc as plsc

EMBED_DIM = 64
ROW_LANES = 128
NUM_CORES = 2
NUM_SUBCORES = 16
NUM_WORKERS = NUM_CORES * NUM_SUBCORES
CHUNK = 256
PACK_COLS = 8192


def _tc_pack(weight_t, vocab):
    num_blocks = pl.cdiv(vocab, PACK_COLS)

    def pack_kernel(wt_ref, out_ref):
        out_ref[:, :EMBED_DIM] = wt_ref[...].T

    return pl.pallas_call(
        pack_kernel,
        grid=(num_blocks,),
        in_specs=[pl.BlockSpec((EMBED_DIM, PACK_COLS), lambda i: (0, i))],
        out_specs=pl.BlockSpec((PACK_COLS, ROW_LANES), lambda i: (i, 0)),
        out_shape=jax.ShapeDtypeStruct((num_blocks * PACK_COLS, ROW_LANES),
                                       jnp.float32),
    )(weight_t)


def _sc_gather(table, idx, num_indices, chunk=CHUNK):
    mesh = plsc.VectorSubcoreMesh(core_axis_name="c", subcore_axis_name="s")
    per_worker = num_indices // NUM_WORKERS
    num_chunks = per_worker // chunk

    @functools.partial(
        pl.kernel,
        mesh=mesh,
        out_type=jax.ShapeDtypeStruct((num_indices, ROW_LANES), jnp.float32),
        scratch_types=[
            pltpu.VMEM((chunk,), jnp.int32),
            pltpu.VMEM((chunk,), jnp.int32),
            pltpu.VMEM((chunk, ROW_LANES), jnp.float32),
            pltpu.VMEM((chunk, ROW_LANES), jnp.float32),
            pltpu.SemaphoreType.DMA,
            pltpu.SemaphoreType.DMA,
            pltpu.SemaphoreType.DMA,
        ],
    )
    def sc_kernel(table_hbm, idx_hbm, out_hbm, idx_v0, idx_v1,
                  rows_v0, rows_v1, gsem, wsem0, wsem1):
        wid = lax.axis_index("s") * NUM_CORES + lax.axis_index("c")
        base = wid * per_worker
        idx_v = (idx_v0, idx_v1)
        rows_v = (rows_v0, rows_v1)
        wsem = (wsem0, wsem1)

        def step(c, b, first):
            start = base + c * chunk
            pltpu.sync_copy(idx_hbm.at[pl.ds(start, chunk)], idx_v[b])
            if not first:
                @pl.when(c >= 2)
                def _():
                    pltpu.make_async_copy(
                        rows_v[b], out_hbm.at[pl.ds(start, chunk)],
                        wsem[b]).wait()
            pltpu.async_copy(table_hbm.at[idx_v[b]], rows_v[b], gsem).wait()
            pltpu.async_copy(rows_v[b], out_hbm.at[pl.ds(start, chunk)],
                             wsem[b])

        @pl.loop(0, num_chunks, step=2)
        def _(c):
            step(c, 0, False)
            step(c + 1, 1, False)

        pltpu.make_async_copy(
            rows_v0, out_hbm.at[pl.ds(base, chunk)], wsem0).wait()
        pltpu.make_async_copy(
            rows_v1, out_hbm.at[pl.ds(base, chunk)], wsem1).wait()

    return sc_kernel(table, idx)


def kernel(batchinput, weight):
    batch, seq = batchinput.shape
    num_indices = batch * seq
    idx_flat = batchinput.reshape(num_indices)

    table = _tc_pack(weight.T, weight.shape[0])
    rows = _sc_gather(table, idx_flat, num_indices)
    return rows[:, :EMBED_DIM].reshape(batch, seq, EMBED_DIM)


# speedup vs baseline: 1.9580x; 1.0509x over previous
import functools
import jax
import jax.numpy as jnp
from jax import lax
from jax.experimental import pallas as pl
from jax.experimental.pallas import tpu as pltpu
from jax.experimental.pallas import tpu_sc as plsc

EMBED_DIM = 64
ROW_LANES = 128
NUM_CORES = 2
NUM_SUBCORES = 16
NUM_WORKERS = NUM_CORES * NUM_SUBCORES
CHUNK = 256
PACK_COLS = 8192


def _tc_pack(weight_t, vocab):
    num_blocks = pl.cdiv(vocab, PACK_COLS)

    def pack_kernel(wt_ref, out_ref):
        out_ref[:, :EMBED_DIM] = wt_ref[...].T

    return pl.pallas_call(
        pack_kernel,
        grid=(num_blocks,),
        in_specs=[pl.BlockSpec((EMBED_DIM, PACK_COLS), lambda i: (0, i))],
        out_specs=pl.BlockSpec((PACK_COLS, ROW_LANES), lambda i: (i, 0)),
        out_shape=jax.ShapeDtypeStruct((num_blocks * PACK_COLS, ROW_LANES),
                                       jnp.float32),
    )(weight_t)


def _sc_gather(table, idx, num_indices, chunk=CHUNK):
    mesh = plsc.VectorSubcoreMesh(core_axis_name="c", subcore_axis_name="s")
    per_worker = num_indices // NUM_WORKERS
    num_chunks = per_worker // chunk

    @functools.partial(
        pl.kernel,
        mesh=mesh,
        out_type=jax.ShapeDtypeStruct((num_indices, ROW_LANES), jnp.float32),
        scratch_types=[
            pltpu.VMEM((chunk,), jnp.int32),
            pltpu.VMEM((chunk,), jnp.int32),
            pltpu.VMEM((chunk, ROW_LANES), jnp.float32),
            pltpu.VMEM((chunk, ROW_LANES), jnp.float32),
            pltpu.SemaphoreType.DMA,
            pltpu.SemaphoreType.DMA,
            pltpu.SemaphoreType.DMA,
            pltpu.SemaphoreType.DMA,
        ],
    )
    def sc_kernel(table_hbm, idx_hbm, out_hbm, idx_v0, idx_v1,
                  rows_v0, rows_v1, gsem0, gsem1, wsem0, wsem1):
        wid = lax.axis_index("s") * NUM_CORES + lax.axis_index("c")
        base = wid * per_worker
        idx_v = (idx_v0, idx_v1)
        rows_v = (rows_v0, rows_v1)
        gsem = (gsem0, gsem1)
        wsem = (wsem0, wsem1)

        def start_gather(c, b):
            start = base + c * chunk
            pltpu.sync_copy(idx_hbm.at[pl.ds(start, chunk)], idx_v[b])

            @pl.when(c >= 2)
            def _():
                pltpu.make_async_copy(
                    rows_v[b], out_hbm.at[pl.ds(start, chunk)],
                    wsem[b]).wait()
            pltpu.async_copy(table_hbm.at[idx_v[b]], rows_v[b], gsem[b])

        def finish(c, b):
            start = base + c * chunk
            pltpu.make_async_copy(table_hbm.at[idx_v[b]], rows_v[b],
                                  gsem[b]).wait()
            pltpu.async_copy(rows_v[b], out_hbm.at[pl.ds(start, chunk)],
                             wsem[b])

        @pl.loop(0, num_chunks, step=2)
        def _(c):
            start_gather(c, 0)
            start_gather(c + 1, 1)
            finish(c, 0)
            finish(c + 1, 1)

        pltpu.make_async_copy(
            rows_v0, out_hbm.at[pl.ds(base, chunk)], wsem0).wait()
        pltpu.make_async_copy(
            rows_v1, out_hbm.at[pl.ds(base, chunk)], wsem1).wait()

    return sc_kernel(table, idx)


def kernel(batchinput, weight):
    batch, seq = batchinput.shape
    num_indices = batch * seq
    idx_flat = batchinput.reshape(num_indices)

    table = _tc_pack(weight.T, weight.shape[0])
    rows = _sc_gather(table, idx_flat, num_indices)
    return rows[:, :EMBED_DIM].reshape(batch, seq, EMBED_DIM)


# speedup vs baseline: 1.9583x; 1.0001x over previous
import functools
import jax
import jax.numpy as jnp
from jax import lax
from jax.experimental import pallas as pl
from jax.experimental.pallas import tpu as pltpu
from jax.experimental.pallas import tpu_sc as plsc

EMBED_DIM = 64
ROW_LANES = 128
NUM_CORES = 2
NUM_SUBCORES = 16
NUM_WORKERS = NUM_CORES * NUM_SUBCORES
CHUNK = 400
PACK_COLS = 8192


def _tc_pack(weight_t, vocab):
    num_blocks = pl.cdiv(vocab, PACK_COLS)

    def pack_kernel(wt_ref, out_ref):
        out_ref[:, :EMBED_DIM] = wt_ref[...].T

    return pl.pallas_call(
        pack_kernel,
        grid=(num_blocks,),
        in_specs=[pl.BlockSpec((EMBED_DIM, PACK_COLS), lambda i: (0, i))],
        out_specs=pl.BlockSpec((PACK_COLS, ROW_LANES), lambda i: (i, 0)),
        out_shape=jax.ShapeDtypeStruct((num_blocks * PACK_COLS, ROW_LANES),
                                       jnp.float32),
    )(weight_t)


def _sc_gather(table, idx, num_indices, chunk=CHUNK):
    mesh = plsc.VectorSubcoreMesh(core_axis_name="c", subcore_axis_name="s")
    per_worker = num_indices // NUM_WORKERS
    num_chunks = per_worker // chunk

    @functools.partial(
        pl.kernel,
        mesh=mesh,
        out_type=jax.ShapeDtypeStruct((num_indices, ROW_LANES), jnp.float32),
        scratch_types=[
            pltpu.VMEM((chunk,), jnp.int32),
            pltpu.VMEM((chunk,), jnp.int32),
            pltpu.VMEM((chunk, ROW_LANES), jnp.float32),
            pltpu.VMEM((chunk, ROW_LANES), jnp.float32),
            pltpu.SemaphoreType.DMA,
            pltpu.SemaphoreType.DMA,
            pltpu.SemaphoreType.DMA,
            pltpu.SemaphoreType.DMA,
        ],
    )
    def sc_kernel(table_hbm, idx_hbm, out_hbm, idx_v0, idx_v1,
                  rows_v0, rows_v1, gsem0, gsem1, wsem0, wsem1):
        wid = lax.axis_index("s") * NUM_CORES + lax.axis_index("c")
        base = wid * per_worker
        idx_v = (idx_v0, idx_v1)
        rows_v = (rows_v0, rows_v1)
        gsem = (gsem0, gsem1)
        wsem = (wsem0, wsem1)

        def start_gather(c, b):
            start = base + c * chunk
            pltpu.sync_copy(idx_hbm.at[pl.ds(start, chunk)], idx_v[b])

            @pl.when(c >= 2)
            def _():
                pltpu.make_async_copy(
                    rows_v[b], out_hbm.at[pl.ds(start, chunk)],
                    wsem[b]).wait()
            pltpu.async_copy(table_hbm.at[idx_v[b]], rows_v[b], gsem[b])

        def finish(c, b):
            start = base + c * chunk
            pltpu.make_async_copy(table_hbm.at[idx_v[b]], rows_v[b],
                                  gsem[b]).wait()
            pltpu.async_copy(rows_v[b], out_hbm.at[pl.ds(start, chunk)],
                             wsem[b])

        @pl.loop(0, num_chunks, step=2)
        def _(c):
            start_gather(c, 0)
            start_gather(c + 1, 1)
            finish(c, 0)
            finish(c + 1, 1)

        pltpu.make_async_copy(
            rows_v0, out_hbm.at[pl.ds(base, chunk)], wsem0).wait()
        pltpu.make_async_copy(
            rows_v1, out_hbm.at[pl.ds(base, chunk)], wsem1).wait()

    return sc_kernel(table, idx)


def kernel(batchinput, weight):
    batch, seq = batchinput.shape
    num_indices = batch * seq
    idx_flat = batchinput.reshape(num_indices)

    table = _tc_pack(weight.T, weight.shape[0])
    rows = _sc_gather(table, idx_flat, num_indices)
    return rows[:, :EMBED_DIM].reshape(batch, seq, EMBED_DIM)
